# baseline (speedup 1.0000x reference)
import functools

import jax
import jax.numpy as jnp
from jax import lax
from jax.experimental import pallas as pl
from jax.experimental.pallas import tpu as pltpu
from jax.experimental.pallas import tpu_sc as plsc

CHUNK = 128
NBUF = 12
LAG = 4


@functools.lru_cache(maxsize=None)
def _make_gather(n_rows_total, dim, cpw, nc, ns):
    nw = nc * ns
    mesh = plsc.VectorSubcoreMesh(
        core_axis_name="c", subcore_axis_name="s", num_cores=nc,
        num_subcores=ns)

    @functools.partial(
        pl.kernel,
        out_type=jax.ShapeDtypeStruct((n_rows_total * CHUNK, dim),
                                      jnp.float32),
        mesh=mesh,
        compiler_params=pltpu.CompilerParams(use_tc_tiling_on_sc=False),
        scratch_types=[
            pltpu.VMEM((cpw, CHUNK), jnp.int32),
            pltpu.VMEM((NBUF, CHUNK, dim), jnp.float32),
            pltpu.SemaphoreType.DMA((NBUF,)),
            pltpu.SemaphoreType.DMA((NBUF,)),
        ],
    )
    def gather_kernel(idx_hbm, table_hbm, out_hbm, idx_v, rows_v, sem_g,
                      sem_o):
        wid = lax.axis_index("s") * nc + lax.axis_index("c")
        row0 = wid * cpw
        pltpu.sync_copy(idx_hbm.at[pl.ds(row0, cpw)], idx_v)

        for b in range(NBUF):
            pltpu.async_copy(table_hbm.at[idx_v.at[b]], rows_v.at[b],
                             sem_g.at[b])

        def step(j, carry):
            jmod = lax.rem(j, NBUF)
            out_slice = out_hbm.at[pl.ds((row0 + j) * CHUNK, CHUNK)]
            pltpu.make_async_copy(
                table_hbm.at[idx_v.at[j]], rows_v.at[jmod],
                sem_g.at[jmod]).wait()
            pltpu.async_copy(rows_v.at[jmod], out_slice, sem_o.at[jmod])

            @pl.when(jnp.logical_and(j >= LAG, j + NBUF - LAG < cpw))
            def _():
                b2 = lax.rem(j - LAG, NBUF)
                pltpu.make_async_copy(rows_v.at[b2], out_slice,
                                      sem_o.at[b2]).wait()
                pltpu.async_copy(table_hbm.at[idx_v.at[j + NBUF - LAG]],
                                 rows_v.at[b2], sem_g.at[b2])

            return carry

        lax.fori_loop(0, cpw, step, 0)

        for b in range(NBUF):
            pltpu.make_async_copy(
                rows_v.at[b],
                out_hbm.at[pl.ds(row0 * CHUNK, CHUNK)], sem_o.at[b]).wait()

    return gather_kernel


def kernel(x, table):
    dim = table.shape[1]
    orig_shape = x.shape
    flat = x.reshape(-1).astype(jnp.int32)
    b = flat.shape[0]
    info = plsc.get_sparse_core_info()
    nc, ns = info.num_cores, info.num_subcores
    nw = nc * ns
    per_call = nw * CHUNK
    b_pad = ((b + per_call - 1) // per_call) * per_call
    if b_pad != b:
        flat = jnp.pad(flat, (0, b_pad - b))
    cpw = b_pad // per_call
    idx2d = flat.reshape(cpw * nw, CHUNK)
    out = _make_gather(cpw * nw, dim, cpw, nc, ns)(idx2d, table)
    if b_pad != b:
        out = out[:b]
    return out.reshape(orig_shape + (dim,))


# speedup vs baseline: 1.4281x; 1.4281x over previous
import functools

import jax
import jax.numpy as jnp
from jax import lax
from jax.experimental import pallas as pl
from jax.experimental.pallas import tpu as pltpu
from jax.experimental.pallas import tpu_sc as plsc

CHUNK = 128
NBUF = 8
LAG = 2
PAD = 128


@functools.lru_cache(maxsize=None)
def _make_gather(vocab2, dim, cpw, nc, ns):
    nw = nc * ns
    mesh = plsc.VectorSubcoreMesh(
        core_axis_name="c", subcore_axis_name="s", num_cores=nc,
        num_subcores=ns)

    @functools.partial(
        pl.kernel,
        out_type=jax.ShapeDtypeStruct((nw * cpw * CHUNK, PAD), jnp.float32),
        mesh=mesh,
        compiler_params=pltpu.CompilerParams(use_tc_tiling_on_sc=False),
        scratch_types=[
            pltpu.VMEM((cpw, CHUNK), jnp.int32),
            pltpu.VMEM((NBUF, CHUNK, dim), jnp.float32),
            pltpu.SemaphoreType.DMA((NBUF,)),
            pltpu.SemaphoreType.DMA((NBUF,)),
        ],
    )
    def gather_kernel(idx_hbm, table_hbm, out_hbm, idx_v, rows_v, sem_g,
                      sem_o):
        wid = lax.axis_index("s") * nc + lax.axis_index("c")
        row0 = wid * cpw
        pltpu.sync_copy(idx_hbm.at[pl.ds(row0, cpw)], idx_v)

        for b in range(NBUF):
            pltpu.async_copy(table_hbm.at[idx_v.at[b]], rows_v.at[b],
                             sem_g.at[b])

        def step(j, carry):
            jmod = lax.rem(j, NBUF)
            out_slice = out_hbm.at[pl.ds((row0 + j) * CHUNK, CHUNK),
                                   pl.ds(0, dim)]
            pltpu.make_async_copy(
                table_hbm.at[idx_v.at[j]], rows_v.at[jmod],
                sem_g.at[jmod]).wait()
            pltpu.async_copy(rows_v.at[jmod], out_slice, sem_o.at[jmod])

            @pl.when(jnp.logical_and(j >= LAG, j + NBUF - LAG < cpw))
            def _():
                b2 = lax.rem(j - LAG, NBUF)
                pltpu.make_async_copy(rows_v.at[b2], out_slice,
                                      sem_o.at[b2]).wait()
                pltpu.async_copy(table_hbm.at[idx_v.at[j + NBUF - LAG]],
                                 rows_v.at[b2], sem_g.at[b2])

            return carry

        lax.fori_loop(0, cpw, step, 0)

        for b in range(NBUF):
            pltpu.make_async_copy(
                rows_v.at[b],
                out_hbm.at[pl.ds(row0 * CHUNK, CHUNK), pl.ds(0, dim)],
                sem_o.at[b]).wait()

    return gather_kernel


def kernel(x, table):
    vocab, dim = table.shape
    orig_shape = x.shape
    flat = x.reshape(-1).astype(jnp.int32)
    b = flat.shape[0]
    info = plsc.get_sparse_core_info()
    nc, ns = info.num_cores, info.num_subcores
    nw = nc * ns
    per_call = nw * CHUNK
    b_pad = ((b + per_call - 1) // per_call) * per_call
    if b_pad != b:
        flat = jnp.pad(flat, (0, b_pad - b))
    cpw = b_pad // per_call
    idx2d = (flat * (PAD // dim)).reshape(cpw * nw, CHUNK)
    table_pad = jnp.pad(table, ((0, 0), (0, PAD - dim)))
    table2 = table_pad.reshape(vocab * (PAD // dim), dim)
    out128 = _make_gather(table2.shape[0], dim, cpw, nc, ns)(idx2d, table2)
    out = out128.reshape(b_pad // CHUNK * CHUNK, PAD)[:b, :dim]
    return out.reshape(orig_shape + (dim,))
